# baseline (speedup 1.0000x reference)
import functools

import jax
import jax.numpy as jnp
from jax import lax
from jax.experimental import pallas as pl
from jax.experimental.pallas import tpu as pltpu
from jax.experimental.pallas import tpu_sc as plsc

NOISE_RATIO = 0.1
NOISE_PROB = 0.5
MAX_STEP = 30

_H = 512
_W = 512
_P = _H * _W
_N = int(NOISE_RATIO * _P)
_NPAD = ((_N + 127) // 128) * 128

_NW = 32
_RW = _H // _NW
_LANES = 16

_sc_mesh = plsc.VectorSubcoreMesh(core_axis_name="c", subcore_axis_name="s")


@functools.partial(
    pl.kernel,
    mesh=_sc_mesh,
    out_type=jax.ShapeDtypeStruct((_H, _W), jnp.float32),
    scratch_types=[
        pltpu.VMEM((_NPAD,), jnp.int32),
        pltpu.VMEM((_NPAD,), jnp.float32),
        pltpu.VMEM((_RW, _W), jnp.float32),
    ],
    compiler_params=pltpu.CompilerParams(
        needs_layout_passes=False, use_tc_tiling_on_sc=True),
)
def _mask_build(idx_hbm, vals_hbm, out_hbm, idx_v, vals_v, maskb):
    wid = lax.axis_index("s") * 2 + lax.axis_index("c")
    r0 = wid * _RW

    pltpu.sync_copy(idx_hbm, idx_v)
    pltpu.sync_copy(vals_hbm, vals_v)

    ones = jnp.full((_LANES,), 1.0, jnp.float32)

    for r in range(_RW):
        @plsc.parallel_loop(0, _W // _LANES, unroll=8)
        def _init(i):
            maskb[r, pl.ds(i * _LANES, _LANES)] = ones

    lov = jnp.full((_LANES,), r0 * _W, jnp.int32)
    hiv = lov + _RW * _W
    zero = jnp.zeros((_LANES,), jnp.int32)

    @plsc.parallel_loop(0, _NPAD // _LANES, unroll=4)
    def _scatter(i):
        idx = idx_v[pl.ds(i * _LANES, _LANES)]
        v = vals_v[pl.ds(i * _LANES, _LANES)]
        m = (idx >= lov) & (idx < hiv)
        local = jnp.where(m, idx - lov, zero)
        lr = lax.shift_right_logical(local, 9)
        col = lax.bitwise_and(local, jnp.full((_LANES,), _W - 1, jnp.int32))
        plsc.store_scatter(maskb, [lr, col], v, mask=m)

    pltpu.sync_copy(maskb, out_hbm.at[pl.ds(r0, _RW), :])


def _tc_body(img_ref, mask_ref, out_ref):
    out_ref[...] = jnp.clip(img_ref[...] * mask_ref[...][None, :, :], 0.0, 1.0)


def kernel(marked_img, now_step):
    B, C, H, W = marked_img.shape
    num_noisy_pixels = _N

    key = jax.random.key(42)
    kp, kn = jax.random.split(key)
    indices = jax.random.permutation(kp, H * W)[:num_noisy_pixels]
    indices = indices.astype(jnp.int32)
    random_noise = jax.random.uniform(kn, (num_noisy_pixels,), dtype=jnp.float32)
    base_vals = jnp.where(random_noise < NOISE_PROB, 1.0, 0.0).astype(jnp.float32)

    pad = _NPAD - num_noisy_pixels
    idx_full = jnp.concatenate([indices, jnp.full((pad,), _P, jnp.int32)])
    base_full = jnp.concatenate([base_vals, jnp.ones((pad,), jnp.float32)])

    noise_ratio_t = jnp.minimum(now_step / MAX_STEP, 1.0) * NOISE_RATIO
    num_noisy_pixels_t = noise_ratio_t * H * W
    vals_full = jnp.where(
        jnp.arange(_NPAD) < num_noisy_pixels_t, base_full, 1.0
    ).astype(jnp.float32)

    mask2d = _mask_build(idx_full, vals_full)

    img3 = marked_img.reshape(B * C, H, W)
    blk = 6
    out3 = pl.pallas_call(
        _tc_body,
        grid=(B * C // blk,),
        in_specs=[
            pl.BlockSpec((blk, H, W), lambda i: (i, 0, 0)),
            pl.BlockSpec((H, W), lambda i: (0, 0)),
        ],
        out_specs=pl.BlockSpec((blk, H, W), lambda i: (i, 0, 0)),
        out_shape=jax.ShapeDtypeStruct((B * C, H, W), jnp.float32),
    )(img3, mask2d)
    return out3.reshape(B, C, H, W)


# speedup vs baseline: 1.0018x; 1.0018x over previous
import functools

import jax
import jax.numpy as jnp
from jax import lax
from jax.experimental import pallas as pl
from jax.experimental.pallas import tpu as pltpu
from jax.experimental.pallas import tpu_sc as plsc

NOISE_RATIO = 0.1
NOISE_PROB = 0.5
MAX_STEP = 30

_H = 512
_W = 512
_P = _H * _W
_N = int(NOISE_RATIO * _P)
_NPAD = ((_N + 127) // 128) * 128

_NW = 32
_RW = _H // _NW
_LANES = 16

_sc_mesh = plsc.VectorSubcoreMesh(core_axis_name="c", subcore_axis_name="s")


@functools.partial(
    pl.kernel,
    mesh=_sc_mesh,
    out_type=jax.ShapeDtypeStruct((_H, _W), jnp.float32),
    scratch_types=[
        pltpu.VMEM((_NPAD,), jnp.int32),
        pltpu.VMEM((_NPAD,), jnp.float32),
        pltpu.VMEM((_RW, _W), jnp.float32),
    ],
    compiler_params=pltpu.CompilerParams(
        needs_layout_passes=False, use_tc_tiling_on_sc=True),
)
def _mask_build(idx_hbm, vals_hbm, out_hbm, idx_v, vals_v, maskb):
    wid = lax.axis_index("s") * 2 + lax.axis_index("c")
    r0 = wid * _RW

    pltpu.sync_copy(idx_hbm, idx_v)
    pltpu.sync_copy(vals_hbm, vals_v)

    ones = jnp.full((_LANES,), 1.0, jnp.float32)

    for r in range(_RW):
        @plsc.parallel_loop(0, _W // _LANES, unroll=8)
        def _init(i):
            maskb[r, pl.ds(i * _LANES, _LANES)] = ones

    lov = jnp.full((_LANES,), r0 * _W, jnp.int32)
    hiv = lov + _RW * _W
    zero = jnp.zeros((_LANES,), jnp.int32)

    @plsc.parallel_loop(0, _NPAD // _LANES, unroll=4)
    def _scatter(i):
        idx = idx_v[pl.ds(i * _LANES, _LANES)]
        v = vals_v[pl.ds(i * _LANES, _LANES)]
        m = (idx >= lov) & (idx < hiv)
        local = jnp.where(m, idx - lov, zero)
        lr = lax.shift_right_logical(local, 9)
        col = lax.bitwise_and(local, jnp.full((_LANES,), _W - 1, jnp.int32))
        plsc.store_scatter(maskb, [lr, col], v, mask=m)

    pltpu.sync_copy(maskb, out_hbm.at[pl.ds(r0, _RW), :])


def _tc_body(img_ref, mask_ref, out_ref):
    out_ref[...] = jnp.clip(img_ref[...] * mask_ref[...][None, :, :], 0.0, 1.0)


def kernel(marked_img, now_step):
    B, C, H, W = marked_img.shape
    num_noisy_pixels = _N

    key = jax.random.key(42)
    kp, kn = jax.random.split(key)
    indices = jax.random.permutation(kp, H * W)[:num_noisy_pixels]
    indices = indices.astype(jnp.int32)
    random_noise = jax.random.uniform(kn, (num_noisy_pixels,), dtype=jnp.float32)
    base_vals = jnp.where(random_noise < NOISE_PROB, 1.0, 0.0).astype(jnp.float32)

    pad = _NPAD - num_noisy_pixels
    idx_full = jnp.concatenate([indices, jnp.full((pad,), _P, jnp.int32)])
    base_full = jnp.concatenate([base_vals, jnp.ones((pad,), jnp.float32)])

    noise_ratio_t = jnp.minimum(now_step / MAX_STEP, 1.0) * NOISE_RATIO
    num_noisy_pixels_t = noise_ratio_t * H * W
    vals_full = jnp.where(
        jnp.arange(_NPAD) < num_noisy_pixels_t, base_full, 1.0
    ).astype(jnp.float32)

    mask2d = _mask_build(idx_full, vals_full)

    img3 = marked_img.reshape(B * C, H, W)
    blk = 12
    out3 = pl.pallas_call(
        _tc_body,
        grid=(B * C // blk,),
        in_specs=[
            pl.BlockSpec((blk, H, W), lambda i: (i, 0, 0)),
            pl.BlockSpec((H, W), lambda i: (0, 0)),
        ],
        out_specs=pl.BlockSpec((blk, H, W), lambda i: (i, 0, 0)),
        out_shape=jax.ShapeDtypeStruct((B * C, H, W), jnp.float32),
    )(img3, mask2d)
    return out3.reshape(B, C, H, W)
